# baseline (speedup 1.0000x reference)
import functools

import jax
import jax.numpy as jnp
from jax import lax
from jax.experimental import pallas as pl
from jax.experimental.pallas import tpu as pltpu
from jax.experimental.pallas import tpu_sc as plsc

N = 100000
E = 3200000
F_IN = 128
F_H = 16

NC = 2
NS = 16
NW = NC * NS

B = 512
CHUNKS = 196
NPAIR = CHUNKS // 2
E_PAD = CHUNKS * B * NW
N_PAD = 100352
PAD_ROWS = N_PAD - N
NP8 = N_PAD // 8
RPS = N_PAD // NS
ZCH = 64
ZROWS = RPS // ZCH

BLK = 2048
BLKP = BLK // 8


def _mesh():
    return plsc.VectorSubcoreMesh(core_axis_name="c", subcore_axis_name="s")


def _fill_zero_rows(zbuf, nrows):
    def body(i, carry):
        zbuf[i, :] = jnp.zeros((16,), jnp.float32)
        return carry

    lax.fori_loop(0, nrows, body, 0)


@functools.partial(
    pl.kernel,
    out_type=jax.ShapeDtypeStruct((NC, N_PAD, F_H), jnp.float32),
    mesh=_mesh(),
    scratch_types=[
        pltpu.VMEM_SHARED((N_PAD, F_H), jnp.float32),
        pltpu.VMEM((2, 2, B), jnp.int32),
        pltpu.VMEM((2, B, F_H), jnp.float32),
        pltpu.VMEM((ZROWS, F_H), jnp.float32),
        pltpu.SemaphoreType.DMA,
        pltpu.SemaphoreType.DMA,
    ],
    compiler_params=pltpu.CompilerParams(use_tc_tiling_on_sc=False),
)
def _prop16(epk_hbm, g_hbm, out_hbm, acc, ebuf, rows, zbuf, semg, sems):
    c = lax.axis_index("c")
    s = lax.axis_index("s")
    w = s * NC + c

    _fill_zero_rows(zbuf, ZROWS)
    for z in range(ZCH):
        pltpu.sync_copy(zbuf, acc.at[pl.ds(s * RPS + z * ZROWS, ZROWS)])
    plsc.subcore_barrier()

    def stage_fire(b, chunk):
        pltpu.sync_copy(epk_hbm.at[w, pl.ds(2 * chunk, 2)], ebuf.at[b])
        pltpu.async_copy(g_hbm.at[ebuf.at[b, 0]], rows.at[b], semg)

    def wait_gather(b):
        pltpu.make_async_copy(g_hbm.at[ebuf.at[b, 0]], rows.at[b],
                              semg).wait()

    def fire_scatter(b):
        pltpu.async_copy(rows.at[b], acc.at[ebuf.at[b, 1]], sems, add=True)

    def wait_scatter(b):
        pltpu.make_async_copy(rows.at[b], acc.at[ebuf.at[b, 1]], sems).wait()

    stage_fire(0, 0)
    stage_fire(1, 1)

    def body(it, carry):
        a = 2 * it
        wait_gather(0)
        fire_scatter(0)
        wait_gather(1)
        fire_scatter(1)
        wait_scatter(0)
        stage_fire(0, jnp.minimum(a + 2, CHUNKS - 1))
        wait_scatter(1)
        stage_fire(1, jnp.minimum(a + 3, CHUNKS - 1))
        return carry

    lax.fori_loop(0, NPAIR, body, 0)
    wait_gather(0)
    wait_gather(1)

    plsc.subcore_barrier()
    for z in range(ZCH):
        lo = s * RPS + z * ZROWS
        pltpu.sync_copy(acc.at[pl.ds(lo, ZROWS)], zbuf)
        pltpu.sync_copy(zbuf, out_hbm.at[c, pl.ds(lo, ZROWS)])


@functools.partial(
    pl.kernel,
    out_type=jax.ShapeDtypeStruct((NC, N_PAD, F_H), jnp.float32),
    mesh=_mesh(),
    scratch_types=[
        pltpu.VMEM_SHARED((N_PAD, F_H), jnp.float32),
        pltpu.VMEM((2, 2, B), jnp.int32),
        pltpu.VMEM((B, F_H), jnp.float32),
        pltpu.VMEM((ZROWS, F_H), jnp.float32),
        pltpu.SemaphoreType.DMA,
    ],
    compiler_params=pltpu.CompilerParams(use_tc_tiling_on_sc=False),
)
def _deg16(epk_hbm, out_hbm, acc, ebuf, ones, zbuf, sems):
    c = lax.axis_index("c")
    s = lax.axis_index("s")
    w = s * NC + c

    def ones_body(i, carry):
        ones[i, :] = jnp.ones((16,), jnp.float32)
        return carry

    lax.fori_loop(0, B, ones_body, 0)
    _fill_zero_rows(zbuf, ZROWS)
    for z in range(ZCH):
        pltpu.sync_copy(zbuf, acc.at[pl.ds(s * RPS + z * ZROWS, ZROWS)])
    plsc.subcore_barrier()

    def stage(b, chunk):
        pltpu.sync_copy(epk_hbm.at[w, pl.ds(2 * chunk, 2)], ebuf.at[b])

    def fire_scatter(b):
        pltpu.async_copy(ones, acc.at[ebuf.at[b, 1]], sems, add=True)

    def wait_scatter(b):
        pltpu.make_async_copy(ones, acc.at[ebuf.at[b, 1]], sems).wait()

    stage(0, 0)
    stage(1, 1)

    def body(it, carry):
        a = 2 * it
        fire_scatter(0)
        fire_scatter(1)
        wait_scatter(0)
        stage(0, jnp.minimum(a + 2, CHUNKS - 1))
        wait_scatter(1)
        stage(1, jnp.minimum(a + 3, CHUNKS - 1))
        return carry

    lax.fori_loop(0, NPAIR, body, 0)

    plsc.subcore_barrier()
    for z in range(ZCH):
        lo = s * RPS + z * ZROWS
        pltpu.sync_copy(acc.at[pl.ds(lo, ZROWS)], zbuf)
        pltpu.sync_copy(zbuf, out_hbm.at[c, pl.ds(lo, ZROWS)])


def _dense_matmul1(x, w1big):
    def body(x_ref, w_ref, o_ref):
        xf = x_ref[...].reshape(BLKP, 8 * F_IN)
        o_ref[...] = jnp.dot(xf, w_ref[...],
                             preferred_element_type=jnp.float32)

    return pl.pallas_call(
        body,
        grid=(N_PAD // BLK,),
        in_specs=[
            pl.BlockSpec((BLK, F_IN), lambda i: (i, 0)),
            pl.BlockSpec((8 * F_IN, 128), lambda i: (0, 0)),
        ],
        out_specs=pl.BlockSpec((BLKP, 128), lambda i: (i, 0)),
        out_shape=jax.ShapeDtypeStruct((NP8, 128), jnp.float32),
    )(x, w1big)


def _dense_scale1(h1, deg16p):
    def body(h_ref, d0_ref, d1_ref, g_ref, di_ref):
        dinv = lax.rsqrt(d0_ref[0] + d1_ref[0] + 1.0)
        di_ref[...] = dinv
        g_ref[...] = h_ref[...] * dinv

    return pl.pallas_call(
        body,
        grid=(N_PAD // BLK,),
        in_specs=[
            pl.BlockSpec((BLKP, 128), lambda i: (i, 0)),
            pl.BlockSpec((1, BLKP, 128), lambda i: (0, i, 0)),
            pl.BlockSpec((1, BLKP, 128), lambda i: (1, i, 0)),
        ],
        out_specs=[
            pl.BlockSpec((BLKP, 128), lambda i: (i, 0)),
            pl.BlockSpec((BLKP, 128), lambda i: (i, 0)),
        ],
        out_shape=[
            jax.ShapeDtypeStruct((NP8, 128), jnp.float32),
            jax.ShapeDtypeStruct((NP8, 128), jnp.float32),
        ],
    )(h1, deg16p, deg16p)


def _dense_mid(pp, g_prev, dinv16, bbig, wbig):
    def body(p0_ref, p1_ref, g_ref, di_ref, b_ref, w_ref, o_ref):
        dinv = di_ref[...]
        h = dinv * (p0_ref[0] + p1_ref[0] + g_ref[...]) + b_ref[...]
        h = jnp.maximum(h, 0.0)
        o_ref[...] = (
            jnp.dot(h, w_ref[...], preferred_element_type=jnp.float32) * dinv
        )

    return pl.pallas_call(
        body,
        grid=(N_PAD // BLK,),
        in_specs=[
            pl.BlockSpec((1, BLKP, 128), lambda i: (0, i, 0)),
            pl.BlockSpec((1, BLKP, 128), lambda i: (1, i, 0)),
            pl.BlockSpec((BLKP, 128), lambda i: (i, 0)),
            pl.BlockSpec((BLKP, 128), lambda i: (i, 0)),
            pl.BlockSpec((1, 128), lambda i: (0, 0)),
            pl.BlockSpec((128, 128), lambda i: (0, 0)),
        ],
        out_specs=pl.BlockSpec((BLKP, 128), lambda i: (i, 0)),
        out_shape=jax.ShapeDtypeStruct((NP8, 128), jnp.float32),
    )(pp, pp, g_prev, dinv16, bbig, wbig)


def _dense_last(qq, g3, dinv16, b3big):
    def body(q0_ref, q1_ref, g_ref, di_ref, b_ref, o_ref):
        o_ref[...] = (
            di_ref[...] * (q0_ref[0] + q1_ref[0] + g_ref[...]) + b_ref[...]
        )

    return pl.pallas_call(
        body,
        grid=(N_PAD // BLK,),
        in_specs=[
            pl.BlockSpec((1, BLKP, 128), lambda i: (0, i, 0)),
            pl.BlockSpec((1, BLKP, 128), lambda i: (1, i, 0)),
            pl.BlockSpec((BLKP, 128), lambda i: (i, 0)),
            pl.BlockSpec((BLKP, 128), lambda i: (i, 0)),
            pl.BlockSpec((1, 128), lambda i: (0, 0)),
        ],
        out_specs=pl.BlockSpec((BLKP, 128), lambda i: (i, 0)),
        out_shape=jax.ShapeDtypeStruct((NP8, 128), jnp.float32),
    )(qq, qq, g3, dinv16, b3big)


def kernel(x, edge_index, W1, b1, W2, b2, W3, b3):
    f32 = jnp.float32
    src = edge_index[0].astype(jnp.int32)
    dst = edge_index[1].astype(jnp.int32)

    npad_e = E_PAD - E
    pad_idx = N + (lax.iota(jnp.int32, npad_e) % PAD_ROWS)
    srcp = jnp.concatenate([src, pad_idx]).reshape(NW, CHUNKS, B)
    dstp = jnp.concatenate([dst, pad_idx]).reshape(NW, CHUNKS, B)
    epk = jnp.stack([srcp, dstp], axis=2).reshape(NW, 2 * CHUNKS, B)

    eye8 = jnp.eye(8, dtype=f32)
    w1big = jnp.kron(eye8, W1)
    w2big = jnp.kron(eye8, W2)
    w3big = jnp.kron(eye8, W3 @ jnp.ones((1, F_H), f32))
    b1big = jnp.tile(b1, 8).reshape(1, 128)
    b2big = jnp.tile(b2, 8).reshape(1, 128)
    b3big = jnp.tile(b3, 128).reshape(1, 128)

    h1 = _dense_matmul1(x, w1big)
    deg16 = _deg16(epk)
    g1, dinv16 = _dense_scale1(h1, deg16.reshape(NC, NP8, 128))

    p = _prop16(epk, g1.reshape(N_PAD, F_H))
    g2 = _dense_mid(p.reshape(NC, NP8, 128), g1, dinv16, b1big, w2big)

    p2 = _prop16(epk, g2.reshape(N_PAD, F_H))
    g3 = _dense_mid(p2.reshape(NC, NP8, 128), g2, dinv16, b2big, w3big)

    q = _prop16(epk, g3.reshape(N_PAD, F_H))
    out16 = _dense_last(q.reshape(NC, NP8, 128), g3, dinv16, b3big)

    return out16[: N // 8].reshape(N, F_H)[:, :1]


# speedup vs baseline: 1.0371x; 1.0371x over previous
import functools

import jax
import jax.numpy as jnp
from jax import lax
from jax.experimental import pallas as pl
from jax.experimental.pallas import tpu as pltpu
from jax.experimental.pallas import tpu_sc as plsc

N = 100000
E = 3200000
F_IN = 128
F_H = 16

NC = 2
NS = 16
NW = NC * NS

B = 512
CHUNKS = 196
NPAIR = CHUNKS // 2
E_PAD = CHUNKS * B * NW
N_PAD = 100352
PAD_ROWS = N_PAD - N
NP8 = N_PAD // 8
RPS = N_PAD // NS
ZCH = 64
ZROWS = RPS // ZCH

BLK = 2048
BLKP = BLK // 8


def _mesh():
    return plsc.VectorSubcoreMesh(core_axis_name="c", subcore_axis_name="s")


def _fill_zero_rows(zbuf, nrows):
    def body(i, carry):
        zbuf[i, :] = jnp.zeros((16,), jnp.float32)
        return carry

    lax.fori_loop(0, nrows, body, 0)


@functools.partial(
    pl.kernel,
    out_type=jax.ShapeDtypeStruct((NC, N_PAD, F_H), jnp.float32),
    mesh=_mesh(),
    scratch_types=[
        pltpu.VMEM_SHARED((N_PAD, F_H), jnp.float32),
        pltpu.VMEM((2, 2, B), jnp.int32),
        pltpu.VMEM((2, B, F_H), jnp.float32),
        pltpu.VMEM((ZROWS, F_H), jnp.float32),
        pltpu.SemaphoreType.DMA,
        pltpu.SemaphoreType.DMA,
    ],
    compiler_params=pltpu.CompilerParams(use_tc_tiling_on_sc=False),
)
def _prop16(epk_hbm, g_hbm, out_hbm, acc, ebuf, rows, zbuf, semg, sems):
    c = lax.axis_index("c")
    s = lax.axis_index("s")
    w = s * NC + c

    _fill_zero_rows(zbuf, ZROWS)
    for z in range(ZCH):
        pltpu.sync_copy(zbuf, acc.at[pl.ds(s * RPS + z * ZROWS, ZROWS)])
    plsc.subcore_barrier()

    def stage_fire(b, chunk):
        pltpu.sync_copy(epk_hbm.at[w, pl.ds(2 * chunk, 2)], ebuf.at[b])
        pltpu.async_copy(g_hbm.at[ebuf.at[b, 0]], rows.at[b], semg)

    def wait_gather(b):
        pltpu.make_async_copy(g_hbm.at[ebuf.at[b, 0]], rows.at[b],
                              semg).wait()

    def fire_scatter(b):
        pltpu.async_copy(rows.at[b], acc.at[ebuf.at[b, 1]], sems, add=True)

    def wait_scatter(b):
        pltpu.make_async_copy(rows.at[b], acc.at[ebuf.at[b, 1]], sems).wait()

    stage_fire(0, 0)
    stage_fire(1, 1)

    def body(it, carry):
        a = 2 * it
        wait_gather(0)
        fire_scatter(0)
        wait_gather(1)
        fire_scatter(1)
        wait_scatter(0)
        stage_fire(0, jnp.minimum(a + 2, CHUNKS - 1))
        wait_scatter(1)
        stage_fire(1, jnp.minimum(a + 3, CHUNKS - 1))
        return carry

    lax.fori_loop(0, NPAIR, body, 0)
    wait_gather(0)
    wait_gather(1)

    plsc.subcore_barrier()
    for z in range(ZCH):
        lo = s * RPS + z * ZROWS
        pltpu.sync_copy(acc.at[pl.ds(lo, ZROWS)], zbuf)
        pltpu.sync_copy(zbuf, out_hbm.at[c, pl.ds(lo, ZROWS)])


@functools.partial(
    pl.kernel,
    out_type=jax.ShapeDtypeStruct((NC, N_PAD, F_H), jnp.float32),
    mesh=_mesh(),
    scratch_types=[
        pltpu.VMEM_SHARED((N_PAD, F_H), jnp.float32),
        pltpu.VMEM((2, 2, B), jnp.int32),
        pltpu.VMEM((B, F_H), jnp.float32),
        pltpu.VMEM((ZROWS, F_H), jnp.float32),
        pltpu.SemaphoreType.DMA,
    ],
    compiler_params=pltpu.CompilerParams(use_tc_tiling_on_sc=False),
)
def _deg16(epk_hbm, out_hbm, acc, ebuf, ones, zbuf, sems):
    c = lax.axis_index("c")
    s = lax.axis_index("s")
    w = s * NC + c

    def ones_body(i, carry):
        ones[i, :] = jnp.ones((16,), jnp.float32)
        return carry

    lax.fori_loop(0, B, ones_body, 0)
    _fill_zero_rows(zbuf, ZROWS)
    for z in range(ZCH):
        pltpu.sync_copy(zbuf, acc.at[pl.ds(s * RPS + z * ZROWS, ZROWS)])
    plsc.subcore_barrier()

    def stage(b, chunk):
        pltpu.sync_copy(epk_hbm.at[w, pl.ds(2 * chunk, 2)], ebuf.at[b])

    def fire_scatter(b):
        pltpu.async_copy(ones, acc.at[ebuf.at[b, 1]], sems, add=True)

    def wait_scatter(b):
        pltpu.make_async_copy(ones, acc.at[ebuf.at[b, 1]], sems).wait()

    stage(0, 0)
    stage(1, 1)

    def body(it, carry):
        a = 2 * it
        fire_scatter(0)
        fire_scatter(1)
        wait_scatter(0)
        stage(0, jnp.minimum(a + 2, CHUNKS - 1))
        wait_scatter(1)
        stage(1, jnp.minimum(a + 3, CHUNKS - 1))
        return carry

    lax.fori_loop(0, NPAIR, body, 0)

    plsc.subcore_barrier()
    for z in range(ZCH):
        lo = s * RPS + z * ZROWS
        pltpu.sync_copy(acc.at[pl.ds(lo, ZROWS)], zbuf)
        pltpu.sync_copy(zbuf, out_hbm.at[c, pl.ds(lo, ZROWS)])


def _dense_matmul1(x, w1big):
    def body(x_ref, w_ref, o_ref):
        xf = x_ref[...].reshape(BLKP, 8 * F_IN)
        o_ref[...] = jnp.dot(xf, w_ref[...],
                             preferred_element_type=jnp.float32)

    return pl.pallas_call(
        body,
        grid=(N_PAD // BLK,),
        in_specs=[
            pl.BlockSpec((BLK, F_IN), lambda i: (i, 0)),
            pl.BlockSpec((8 * F_IN, 128), lambda i: (0, 0)),
        ],
        out_specs=pl.BlockSpec((BLKP, 128), lambda i: (i, 0)),
        out_shape=jax.ShapeDtypeStruct((NP8, 128), jnp.float32),
    )(x, w1big)


def _dense_scale1(h1, deg16p):
    def body(h_ref, d0_ref, d1_ref, g_ref, di_ref):
        dinv = lax.rsqrt(d0_ref[0] + d1_ref[0] + 1.0)
        di_ref[...] = dinv
        g_ref[...] = h_ref[...] * dinv

    return pl.pallas_call(
        body,
        grid=(N_PAD // BLK,),
        in_specs=[
            pl.BlockSpec((BLKP, 128), lambda i: (i, 0)),
            pl.BlockSpec((1, BLKP, 128), lambda i: (0, i, 0)),
            pl.BlockSpec((1, BLKP, 128), lambda i: (1, i, 0)),
        ],
        out_specs=[
            pl.BlockSpec((BLKP, 128), lambda i: (i, 0)),
            pl.BlockSpec((BLKP, 128), lambda i: (i, 0)),
        ],
        out_shape=[
            jax.ShapeDtypeStruct((NP8, 128), jnp.float32),
            jax.ShapeDtypeStruct((NP8, 128), jnp.float32),
        ],
    )(h1, deg16p, deg16p)


def _dense_mid(pp, g_prev, dinv16, bbig, wbig):
    def body(p0_ref, p1_ref, g_ref, di_ref, b_ref, w_ref, o_ref):
        dinv = di_ref[...]
        h = dinv * (p0_ref[0] + p1_ref[0] + g_ref[...]) + b_ref[...]
        h = jnp.maximum(h, 0.0)
        o_ref[...] = (
            jnp.dot(h, w_ref[...], preferred_element_type=jnp.float32) * dinv
        )

    return pl.pallas_call(
        body,
        grid=(N_PAD // BLK,),
        in_specs=[
            pl.BlockSpec((1, BLKP, 128), lambda i: (0, i, 0)),
            pl.BlockSpec((1, BLKP, 128), lambda i: (1, i, 0)),
            pl.BlockSpec((BLKP, 128), lambda i: (i, 0)),
            pl.BlockSpec((BLKP, 128), lambda i: (i, 0)),
            pl.BlockSpec((1, 128), lambda i: (0, 0)),
            pl.BlockSpec((128, 128), lambda i: (0, 0)),
        ],
        out_specs=pl.BlockSpec((BLKP, 128), lambda i: (i, 0)),
        out_shape=jax.ShapeDtypeStruct((NP8, 128), jnp.float32),
    )(pp, pp, g_prev, dinv16, bbig, wbig)


def _dense_last(qq, g3, dinv16, b3big, pick):
    def body(q0_ref, q1_ref, g_ref, di_ref, b_ref, p_ref, o_ref):
        v = (
            di_ref[...] * (q0_ref[0] + q1_ref[0] + g_ref[...]) + b_ref[...]
        )
        o_ref[...] = jnp.dot(v, p_ref[...], preferred_element_type=jnp.float32)

    return pl.pallas_call(
        body,
        grid=(N_PAD // BLK,),
        in_specs=[
            pl.BlockSpec((1, BLKP, 128), lambda i: (0, i, 0)),
            pl.BlockSpec((1, BLKP, 128), lambda i: (1, i, 0)),
            pl.BlockSpec((BLKP, 128), lambda i: (i, 0)),
            pl.BlockSpec((BLKP, 128), lambda i: (i, 0)),
            pl.BlockSpec((1, 128), lambda i: (0, 0)),
            pl.BlockSpec((128, 8), lambda i: (0, 0)),
        ],
        out_specs=pl.BlockSpec((BLKP, 8), lambda i: (i, 0)),
        out_shape=jax.ShapeDtypeStruct((NP8, 8), jnp.float32),
    )(qq, qq, g3, dinv16, b3big, pick)


def kernel(x, edge_index, W1, b1, W2, b2, W3, b3):
    f32 = jnp.float32
    src = edge_index[0].astype(jnp.int32)
    dst = edge_index[1].astype(jnp.int32)

    npad_e = E_PAD - E
    pad_idx = N + (lax.iota(jnp.int32, npad_e) % PAD_ROWS)
    srcp = jnp.concatenate([src, pad_idx]).reshape(NW, CHUNKS, B)
    dstp = jnp.concatenate([dst, pad_idx]).reshape(NW, CHUNKS, B)
    epk = jnp.stack([srcp, dstp], axis=2).reshape(NW, 2 * CHUNKS, B)

    eye8 = jnp.eye(8, dtype=f32)
    w1big = jnp.kron(eye8, W1)
    w2big = jnp.kron(eye8, W2)
    w3big = jnp.kron(eye8, W3 @ jnp.ones((1, F_H), f32))
    b1big = jnp.tile(b1, 8).reshape(1, 128)
    b2big = jnp.tile(b2, 8).reshape(1, 128)
    b3big = jnp.tile(b3, 128).reshape(1, 128)

    h1 = _dense_matmul1(x, w1big)
    deg16 = _deg16(epk)
    g1, dinv16 = _dense_scale1(h1, deg16.reshape(NC, NP8, 128))

    p = _prop16(epk, g1.reshape(N_PAD, F_H))
    g2 = _dense_mid(p.reshape(NC, NP8, 128), g1, dinv16, b1big, w2big)

    p2 = _prop16(epk, g2.reshape(N_PAD, F_H))
    g3 = _dense_mid(p2.reshape(NC, NP8, 128), g2, dinv16, b2big, w3big)

    q = _prop16(epk, g3.reshape(N_PAD, F_H))
    pick = (lax.iota(jnp.int32, 128)[:, None]
            == 16 * lax.iota(jnp.int32, 8)[None, :]).astype(f32)
    out8 = _dense_last(q.reshape(NC, NP8, 128), g3, dinv16, b3big, pick)

    return out8.reshape(N_PAD, 1)[:N]
